# baseline (speedup 1.0000x reference)
import functools

import jax
import jax.numpy as jnp
from jax import lax
from jax.experimental import pallas as pl
from jax.experimental.pallas import tpu as pltpu
from jax.experimental.pallas import tpu_sc as plsc

_B, _L, _EMB = 4096, 200, 100
_DP = 112
_NC, _NS = 2, 16
_NW = _NC * _NS
_BTOT = _B * _L
_BPW = _BTOT // _NW
_C = 128
_NCHUNK = _BPW // _C
_CW = _C * _EMB

_mesh = plsc.VectorSubcoreMesh(core_axis_name="c", subcore_axis_name="s")


def _compact(rows_v, cbuf):
    def quad(grp, carry):
        for q in range(4):
            r = grp * 4 + q
            off = grp * 400 + q * 100
            for k in range(7):
                cbuf[pl.ds(off + 16 * k, 16)] = rows_v[r, pl.ds(16 * k, 16)]
        return carry

    lax.fori_loop(0, _C // 4, quad, 0)


@functools.partial(
    pl.kernel,
    out_type=jax.ShapeDtypeStruct((_BTOT * _EMB,), jnp.float32),
    mesh=_mesh,
    scratch_types=[
        pltpu.VMEM((_NCHUNK, _C), jnp.int32),
        pltpu.VMEM((_C, _DP), jnp.float32),
        pltpu.VMEM((_C, _DP), jnp.float32),
        pltpu.VMEM((_CW + 12,), jnp.float32),
        pltpu.VMEM((_CW + 12,), jnp.float32),
        pltpu.SemaphoreType.DMA,
        pltpu.SemaphoreType.DMA,
        pltpu.SemaphoreType.DMA,
        pltpu.SemaphoreType.DMA,
    ],
    compiler_params=pltpu.CompilerParams(use_tc_tiling_on_sc=False),
)
def _gather(idx_hbm, table_hbm, out_hbm, idx_v, rows0, rows1, cb0, cb1,
            gsem0, gsem1, osem0, osem1):
    wid = lax.axis_index("s") * _NC + lax.axis_index("c")
    pltpu.sync_copy(idx_hbm.at[pl.ds(wid * _NCHUNK, _NCHUNK)], idx_v)
    base = wid * _BPW

    rows = (rows0, rows1)
    cbs = (cb0, cb1)
    gsems = (gsem0, gsem1)
    osems = (osem0, osem1)

    for b in range(2):
        pltpu.async_copy(table_hbm.at[idx_v.at[b]], rows[b], gsems[b])

    def pair(go, carry):
        for b in range(2):
            g = go * 2 + b
            pltpu.make_async_copy(
                table_hbm.at[idx_v.at[0]], rows[b], gsems[b]).wait()
            @pl.when(go >= 1)
            def _():
                pltpu.make_async_copy(
                    cbs[b].at[pl.ds(0, _CW)], out_hbm.at[pl.ds(0, _CW)],
                    osems[b]).wait()
            _compact(rows[b], cbs[b])
            pltpu.async_copy(
                cbs[b].at[pl.ds(0, _CW)],
                out_hbm.at[pl.ds((base + g * _C) * _EMB, _CW)], osems[b])
            @pl.when(go < _NCHUNK // 2 - 1)
            def _():
                pltpu.async_copy(
                    table_hbm.at[idx_v.at[g + 2]], rows[b], gsems[b])
        return carry

    lax.fori_loop(0, _NCHUNK // 2, pair, 0)

    for b in range(2):
        pltpu.make_async_copy(
            cbs[b].at[pl.ds(0, _CW)], out_hbm.at[pl.ds(0, _CW)],
            osems[b]).wait()


def kernel(X, glove):
    idx = X.reshape(_NW * _NCHUNK, _C).astype(jnp.int32)
    glove_p = jnp.pad(glove, ((0, 0), (0, _DP - _EMB)))
    out = _gather(idx, glove_p)
    return out.reshape(_B, _L, _EMB)


# speedup vs baseline: 2.0881x; 2.0881x over previous
import functools

import jax
import jax.numpy as jnp
from jax import lax
from jax.experimental import pallas as pl
from jax.experimental.pallas import tpu as pltpu
from jax.experimental.pallas import tpu_sc as plsc

_B, _L, _EMB = 4096, 200, 100
_DP = 128
_NC, _NS = 2, 16
_NW = _NC * _NS
_BTOT = _B * _L
_BPW = _BTOT // _NW
_C = 128
_NCHUNK = _BPW // _C

_mesh = plsc.VectorSubcoreMesh(core_axis_name="c", subcore_axis_name="s")


@functools.partial(
    pl.kernel,
    out_type=jax.ShapeDtypeStruct((_BTOT, _DP), jnp.float32),
    mesh=_mesh,
    scratch_types=[
        pltpu.VMEM((_NCHUNK, _C), jnp.int32),
        pltpu.VMEM((_C, _DP), jnp.float32),
        pltpu.SemaphoreType.DMA,
    ],
    compiler_params=pltpu.CompilerParams(use_tc_tiling_on_sc=True),
)
def _gather(idx_hbm, table_hbm, out_hbm, idx_v, rows_v, sem):
    wid = lax.axis_index("s") * _NC + lax.axis_index("c")
    pltpu.sync_copy(idx_hbm.at[pl.ds(wid * _NCHUNK, _NCHUNK)], idx_v)
    base = wid * _BPW

    def body(g, carry):
        pltpu.async_copy(table_hbm.at[idx_v.at[g]], rows_v, sem).wait()
        pltpu.sync_copy(rows_v, out_hbm.at[pl.ds(base + g * _C, _C)])
        return carry

    lax.fori_loop(0, _NCHUNK, body, 0)


def kernel(X, glove):
    idx = X.reshape(_NW * _NCHUNK, _C).astype(jnp.int32)
    glove_p = jnp.pad(glove, ((0, 0), (0, _DP - _EMB)))
    out = _gather(idx, glove_p)
    return out.reshape(_B, _L, _DP)[:, :, :_EMB]
